# Initial kernel scaffold; baseline (speedup 1.0000x reference)
import functools

import jax
import jax.numpy as jnp
from jax import lax
from jax.experimental import pallas as pl
from jax.experimental.pallas import tpu as pltpu
from jax.experimental.pallas import tpu_sc as plsc

N = 10000
E = 320000
BAG = 16
D = 128
C = 16

NC = 2
NS = 16
NW = NC * NS
L = 16

EMB_BLOCKS = (N * BAG) // 128
EDGE_BLOCKS = E // 128
ROW_BLOCKS = N // 8

_mesh = plsc.VectorSubcoreMesh(core_axis_name="c", subcore_axis_name="s")


def _ceil_div(a, b):
    return (a + b - 1) // b


@functools.partial(
    pl.kernel,
    out_type=(
        jax.ShapeDtypeStruct((N, D), jnp.float32),
        jax.ShapeDtypeStruct((NW, N), jnp.float32),
    ),
    mesh=_mesh,
    scratch_types=[
        pltpu.VMEM((128,), jnp.int32),
        pltpu.VMEM((128, D), jnp.float32),
        pltpu.VMEM((8, D), jnp.float32),
        pltpu.VMEM((128,), jnp.int32),
        pltpu.VMEM((N,), jnp.float32),
        pltpu.SemaphoreType.DMA,
    ],
)
def _sc_emb_deg(x_hbm, dst_hbm, table_hbm, hsum_hbm, degp_hbm,
                idx_v, rows_v, acc_v, didx_v, hist_v, sem):
    c = lax.axis_index("c")
    s = lax.axis_index("s")
    wid = s * NC + c

    ones = jnp.ones((L,), jnp.float32)
    zeros = jnp.zeros((L,), jnp.float32)

    @pl.loop(0, N // L)
    def _(k):
        hist_v[pl.ds(k * L, L)] = zeros

    @pl.loop(0, _ceil_div(EDGE_BLOCKS, NW))
    def _(i):
        b = wid + NW * i

        @pl.when(b < EDGE_BLOCKS)
        def _():
            pltpu.sync_copy(dst_hbm.at[pl.ds(b * 128, 128)], didx_v)
            for j in range(128 // L):
                idx = didx_v[pl.ds(j * L, L)]
                plsc.addupdate_scatter(hist_v, [idx], ones)

    pltpu.sync_copy(hist_v, degp_hbm.at[wid])

    @pl.loop(0, _ceil_div(EMB_BLOCKS, NW))
    def _(i):
        b = wid + NW * i

        @pl.when(b < EMB_BLOCKS)
        def _():
            pltpu.sync_copy(x_hbm.at[pl.ds(b * 128, 128)], idx_v)
            pltpu.async_copy(table_hbm.at[idx_v], rows_v, sem).wait()

            @pl.loop(0, 8)
            def _(n):
                for j in range(D // L):
                    sl = pl.ds(j * L, L)
                    v = rows_v.at[pl.ds(n * BAG, 1)][0, sl]
                    for t in range(1, BAG):
                        v = v + rows_v.at[pl.ds(n * BAG + t, 1)][0, sl]
                    acc_v.at[pl.ds(n, 1)][0, sl] = v

            pltpu.sync_copy(acc_v, hsum_hbm.at[pl.ds(b * 8, 8)])


@functools.partial(
    pl.kernel,
    out_type=jax.ShapeDtypeStruct((NC, N, D), jnp.float32),
    mesh=_mesh,
    scratch_types=[
        pltpu.VMEM((128,), jnp.int32),
        pltpu.VMEM((128,), jnp.int32),
        pltpu.VMEM((128, D), jnp.float32),
        pltpu.VMEM((8, D), jnp.float32),
        pltpu.VMEM_SHARED((N, D), jnp.float32),
        pltpu.SemaphoreType.DMA,
    ],
)
def _sc_scatter(g_hbm, src_hbm, dst_hbm, out_hbm,
                sidx_v, didx_v, rows_v, zero_v, acc_sh, sem):
    c = lax.axis_index("c")
    s = lax.axis_index("s")

    zeros = jnp.zeros((L,), jnp.float32)

    @pl.loop(0, 8)
    def _(r):
        for j in range(D // L):
            zero_v.at[pl.ds(r, 1)][0, pl.ds(j * L, L)] = zeros

    @pl.loop(0, _ceil_div(ROW_BLOCKS, NS))
    def _(i):
        b = s + NS * i

        @pl.when(b < ROW_BLOCKS)
        def _():
            pltpu.sync_copy(zero_v, acc_sh.at[pl.ds(b * 8, 8)])

    plsc.subcore_barrier()

    per_sc = EDGE_BLOCKS // NC

    @pl.loop(0, _ceil_div(per_sc, NS))
    def _(i):
        eb = s + NS * i

        @pl.when(eb < per_sc)
        def _():
            blk = c * per_sc + eb
            pltpu.sync_copy(src_hbm.at[pl.ds(blk * 128, 128)], sidx_v)
            pltpu.sync_copy(dst_hbm.at[pl.ds(blk * 128, 128)], didx_v)
            pltpu.async_copy(g_hbm.at[sidx_v], rows_v, sem).wait()
            pltpu.sync_copy(rows_v, acc_sh.at[didx_v], add=True)

    plsc.subcore_barrier()

    @pl.loop(0, _ceil_div(ROW_BLOCKS, NS))
    def _(i):
        b = s + NS * i

        @pl.when(b < ROW_BLOCKS)
        def _():
            pltpu.sync_copy(acc_sh.at[pl.ds(b * 8, 8)],
                            out_hbm.at[c, pl.ds(b * 8, 8)])


_BM = 1000


def _tc1_body(hs_ref, degp_ref, w_ref, g_ref, dinv_ref):
    h = jnp.maximum(hs_ref[...] * (1.0 / BAG), 0.0)
    deg = jnp.sum(degp_ref[...], axis=0) + 1.0
    dinv = lax.rsqrt(deg)
    hw = jnp.dot(h, w_ref[...], preferred_element_type=jnp.float32)
    g_ref[...] = hw * dinv[:, None]
    dinv_ref[...] = dinv[:, None]


def _tc1(h_sum, degp, W1):
    return pl.pallas_call(
        _tc1_body,
        grid=(N // _BM,),
        in_specs=[
            pl.BlockSpec((_BM, D), lambda i: (i, 0)),
            pl.BlockSpec((NW, _BM), lambda i: (0, i)),
            pl.BlockSpec((D, D), lambda i: (0, 0)),
        ],
        out_specs=[
            pl.BlockSpec((_BM, D), lambda i: (i, 0)),
            pl.BlockSpec((_BM, 1), lambda i: (i, 0)),
        ],
        out_shape=[
            jax.ShapeDtypeStruct((N, D), jnp.float32),
            jax.ShapeDtypeStruct((N, 1), jnp.float32),
        ],
    )(h_sum, degp, W1)


def _tc2_body(p_ref, g_ref, dinv_ref, w_ref, out_ref):
    ssum = p_ref[0] + p_ref[1] + g_ref[...]
    out1 = jnp.maximum(ssum * dinv_ref[...], 0.0)
    hw = jnp.dot(out1, w_ref[...], preferred_element_type=jnp.float32)
    out_ref[...] = hw * dinv_ref[...]


def _tc2(p, g1, dinv, W2):
    return pl.pallas_call(
        _tc2_body,
        grid=(N // _BM,),
        in_specs=[
            pl.BlockSpec((NC, _BM, D), lambda i: (0, i, 0)),
            pl.BlockSpec((_BM, D), lambda i: (i, 0)),
            pl.BlockSpec((_BM, 1), lambda i: (i, 0)),
            pl.BlockSpec((D, D), lambda i: (0, 0)),
        ],
        out_specs=pl.BlockSpec((_BM, D), lambda i: (i, 0)),
        out_shape=jax.ShapeDtypeStruct((N, D), jnp.float32),
    )(p, g1, dinv, W2)


def _tc3_body(p_ref, g_ref, dinv_ref, w_ref, out_ref):
    out2 = (p_ref[0] + p_ref[1] + g_ref[...]) * dinv_ref[...]
    logits = jnp.dot(out2, w_ref[...], preferred_element_type=jnp.float32)
    m = jnp.max(logits, axis=1, keepdims=True)
    e = jnp.exp(logits - m)
    out_ref[...] = e / jnp.sum(e, axis=1, keepdims=True)


def _tc3(p, g2, dinv, Wlin):
    return pl.pallas_call(
        _tc3_body,
        grid=(N // _BM,),
        in_specs=[
            pl.BlockSpec((NC, _BM, D), lambda i: (0, i, 0)),
            pl.BlockSpec((_BM, D), lambda i: (i, 0)),
            pl.BlockSpec((_BM, 1), lambda i: (i, 0)),
            pl.BlockSpec((D, C), lambda i: (0, 0)),
        ],
        out_specs=pl.BlockSpec((_BM, C), lambda i: (i, 0)),
        out_shape=jax.ShapeDtypeStruct((N, C), jnp.float32),
    )(p, g2, dinv, Wlin)


def kernel(x, edge_index, emb_table, W1, W2, Wlin):
    x_flat = x.reshape(-1)
    src = edge_index[0]
    dst = edge_index[1]

    h_sum, degp = _sc_emb_deg(x_flat, dst, emb_table)
    g1, dinv = _tc1(h_sum, degp, W1)
    p1 = _sc_scatter(g1, src, dst)
    g2 = _tc2(p1, g1, dinv, W2)
    p2 = _sc_scatter(g2, src, dst)
    return _tc3(p2, g2, dinv, Wlin)


# speedup vs baseline: 12.4151x; 12.4151x over previous
import dataclasses
import functools

import jax
import jax.numpy as jnp
from jax import lax
from jax.experimental import pallas as pl
from jax.experimental.pallas import tpu as pltpu
from jax.experimental.pallas import tpu_sc as plsc

N = 10000
E = 320000
BAG = 16
D = 128
C = 16

NC = 2
NS = 16
NW = NC * NS
L = 16

EMB_BLOCKS = (N * BAG) // 128
EDGE_BLOCKS = E // 128
ROW_BLOCKS = N // 8

_mesh = plsc.VectorSubcoreMesh(core_axis_name="c", subcore_axis_name="s")

_sc_params = pltpu.CompilerParams()
if "needs_layout_passes" in pltpu.CompilerParams.__dataclass_fields__:
    _sc_params = dataclasses.replace(_sc_params, needs_layout_passes=False)


def _ceil_div(a, b):
    return (a + b - 1) // b


@functools.partial(
    pl.kernel,
    out_type=(
        jax.ShapeDtypeStruct((N, D), jnp.float32),
        jax.ShapeDtypeStruct((NW, N), jnp.float32),
    ),
    mesh=_mesh,
    scratch_types=[
        pltpu.VMEM((128,), jnp.int32),
        pltpu.VMEM((128, D), jnp.float32),
        pltpu.VMEM((8, D), jnp.float32),
        pltpu.VMEM((128,), jnp.int32),
        pltpu.VMEM((N,), jnp.float32),
        pltpu.SemaphoreType.DMA,
    ],
    compiler_params=_sc_params,
)
def _sc_emb_deg(x_hbm, dst_hbm, table_hbm, hsum_hbm, degp_hbm,
                idx_v, rows_v, acc_v, didx_v, hist_v, sem):
    c = lax.axis_index("c")
    s = lax.axis_index("s")
    wid = s * NC + c

    ones = jnp.ones((L,), jnp.float32)
    zeros = jnp.zeros((L,), jnp.float32)

    @pl.loop(0, N // L)
    def _(k):
        hist_v[pl.ds(k * L, L)] = zeros

    @pl.loop(0, _ceil_div(EDGE_BLOCKS, NW))
    def _(i):
        b = wid + NW * i

        @pl.when(b < EDGE_BLOCKS)
        def _():
            pltpu.sync_copy(dst_hbm.at[pl.ds(b * 128, 128)], didx_v)
            for j in range(128 // L):
                idx = didx_v[pl.ds(j * L, L)]
                plsc.addupdate_scatter(hist_v, [idx], ones)

    pltpu.sync_copy(hist_v, degp_hbm.at[wid])

    @pl.loop(0, _ceil_div(EMB_BLOCKS, NW))
    def _(i):
        b = wid + NW * i

        @pl.when(b < EMB_BLOCKS)
        def _():
            pltpu.sync_copy(x_hbm.at[pl.ds(b * 128, 128)], idx_v)
            pltpu.async_copy(table_hbm.at[idx_v], rows_v, sem).wait()

            @pl.loop(0, 8)
            def _(n):
                for j in range(D // L):
                    sl = pl.ds(j * L, L)
                    v = rows_v.at[pl.ds(n * BAG, 1)][0, sl]
                    for t in range(1, BAG):
                        v = v + rows_v.at[pl.ds(n * BAG + t, 1)][0, sl]
                    acc_v.at[pl.ds(n, 1)][0, sl] = v

            pltpu.sync_copy(acc_v, hsum_hbm.at[pl.ds(b * 8, 8)])


@functools.partial(
    pl.kernel,
    out_type=jax.ShapeDtypeStruct((NC, N, D), jnp.float32),
    mesh=_mesh,
    scratch_types=[
        pltpu.VMEM((128,), jnp.int32),
        pltpu.VMEM((128,), jnp.int32),
        pltpu.VMEM((128, D), jnp.float32),
        pltpu.VMEM((8, D), jnp.float32),
        pltpu.VMEM_SHARED((N, D), jnp.float32),
        pltpu.SemaphoreType.DMA,
    ],
    compiler_params=_sc_params,
)
def _sc_scatter(g_hbm, src_hbm, dst_hbm, out_hbm,
                sidx_v, didx_v, rows_v, zero_v, acc_sh, sem):
    c = lax.axis_index("c")
    s = lax.axis_index("s")

    zeros = jnp.zeros((L,), jnp.float32)

    @pl.loop(0, 8)
    def _(r):
        for j in range(D // L):
            zero_v.at[pl.ds(r, 1)][0, pl.ds(j * L, L)] = zeros

    @pl.loop(0, _ceil_div(ROW_BLOCKS, NS))
    def _(i):
        b = s + NS * i

        @pl.when(b < ROW_BLOCKS)
        def _():
            pltpu.sync_copy(zero_v, acc_sh.at[pl.ds(b * 8, 8)])

    plsc.subcore_barrier()

    per_sc = EDGE_BLOCKS // NC

    @pl.loop(0, _ceil_div(per_sc, NS))
    def _(i):
        eb = s + NS * i

        @pl.when(eb < per_sc)
        def _():
            blk = c * per_sc + eb
            pltpu.sync_copy(src_hbm.at[pl.ds(blk * 128, 128)], sidx_v)
            pltpu.sync_copy(dst_hbm.at[pl.ds(blk * 128, 128)], didx_v)
            pltpu.async_copy(g_hbm.at[sidx_v], rows_v, sem).wait()
            pltpu.sync_copy(rows_v, acc_sh.at[didx_v], add=True)

    plsc.subcore_barrier()

    @pl.loop(0, _ceil_div(ROW_BLOCKS, NS))
    def _(i):
        b = s + NS * i

        @pl.when(b < ROW_BLOCKS)
        def _():
            pltpu.sync_copy(acc_sh.at[pl.ds(b * 8, 8)],
                            out_hbm.at[c, pl.ds(b * 8, 8)])


_BM = 1000


def _tc1_body(hs_ref, degp_ref, w_ref, g_ref, dinv_ref):
    h = jnp.maximum(hs_ref[...] * (1.0 / BAG), 0.0)
    deg = jnp.sum(degp_ref[...], axis=1, keepdims=True) + 1.0
    dinv = lax.rsqrt(deg)
    hw = jnp.dot(h, w_ref[...], preferred_element_type=jnp.float32)
    g_ref[...] = hw * dinv
    dinv_ref[...] = dinv


def _tc1(h_sum, degp, W1):
    return pl.pallas_call(
        _tc1_body,
        grid=(N // _BM,),
        in_specs=[
            pl.BlockSpec((_BM, D), lambda i: (i, 0)),
            pl.BlockSpec((_BM, NW), lambda i: (i, 0)),
            pl.BlockSpec((D, D), lambda i: (0, 0)),
        ],
        out_specs=[
            pl.BlockSpec((_BM, D), lambda i: (i, 0)),
            pl.BlockSpec((_BM, 1), lambda i: (i, 0)),
        ],
        out_shape=[
            jax.ShapeDtypeStruct((N, D), jnp.float32),
            jax.ShapeDtypeStruct((N, 1), jnp.float32),
        ],
    )(h_sum, degp, W1)


def _tc2_body(p_ref, g_ref, dinv_ref, w_ref, out_ref):
    ssum = p_ref[0] + p_ref[1] + g_ref[...]
    out1 = jnp.maximum(ssum * dinv_ref[...], 0.0)
    hw = jnp.dot(out1, w_ref[...], preferred_element_type=jnp.float32)
    out_ref[...] = hw * dinv_ref[...]


def _tc2(p, g1, dinv, W2):
    return pl.pallas_call(
        _tc2_body,
        grid=(N // _BM,),
        in_specs=[
            pl.BlockSpec((NC, _BM, D), lambda i: (0, i, 0)),
            pl.BlockSpec((_BM, D), lambda i: (i, 0)),
            pl.BlockSpec((_BM, 1), lambda i: (i, 0)),
            pl.BlockSpec((D, D), lambda i: (0, 0)),
        ],
        out_specs=pl.BlockSpec((_BM, D), lambda i: (i, 0)),
        out_shape=jax.ShapeDtypeStruct((N, D), jnp.float32),
    )(p, g1, dinv, W2)


def _tc3_body(p_ref, g_ref, dinv_ref, w_ref, out_ref):
    out2 = (p_ref[0] + p_ref[1] + g_ref[...]) * dinv_ref[...]
    logits = jnp.dot(out2, w_ref[...], preferred_element_type=jnp.float32)
    m = jnp.max(logits, axis=1, keepdims=True)
    e = jnp.exp(logits - m)
    out_ref[...] = e / jnp.sum(e, axis=1, keepdims=True)


def _tc3(p, g2, dinv, Wlin):
    return pl.pallas_call(
        _tc3_body,
        grid=(N // _BM,),
        in_specs=[
            pl.BlockSpec((NC, _BM, D), lambda i: (0, i, 0)),
            pl.BlockSpec((_BM, D), lambda i: (i, 0)),
            pl.BlockSpec((_BM, 1), lambda i: (i, 0)),
            pl.BlockSpec((D, C), lambda i: (0, 0)),
        ],
        out_specs=pl.BlockSpec((_BM, C), lambda i: (i, 0)),
        out_shape=jax.ShapeDtypeStruct((N, C), jnp.float32),
    )(p, g2, dinv, Wlin)


def kernel(x, edge_index, emb_table, W1, W2, Wlin):
    x_flat = x.reshape(-1)
    src = edge_index[0]
    dst = edge_index[1]

    h_sum, degp = _sc_emb_deg(x_flat, dst, emb_table)
    g1, dinv = _tc1(h_sum, jnp.swapaxes(degp, 0, 1), W1)
    p1 = _sc_scatter(g1, src, dst)
    g2 = _tc2(p1, g1, dinv, W2)
    p2 = _sc_scatter(g2, src, dst)
    return _tc3(p2, g2, dinv, Wlin)
